# baseline (speedup 1.0000x reference)
import jax
import jax.numpy as jnp
from jax.experimental import pallas as pl


def _lrelu(x):
    return jnp.where(x > 0, x, 0.2 * x)


def _knn(x, k):
    xn = jnp.sum(x * x, axis=0, keepdims=True)
    inner = x.T @ x
    pd = xn.T + xn - 2.0 * inner
    _, idx = jax.lax.top_k(-pd, k)
    return idx


def _graph_feat(x, k):
    idx = _knn(x, k)
    xt = x.T
    nb = jnp.take(xt, idx, axis=0)
    ctr = jnp.broadcast_to(xt[:, None, :], nb.shape)
    ef = jnp.concatenate([nb - ctr, ctr], axis=-1)
    return jnp.transpose(ef, (2, 0, 1))


def _edge_conv(x, W, g, b, k):
    ef = _graph_feat(x, k)
    y = jnp.einsum('oc,cnk->onk', W, ef)
    y = _lrelu(y * g[:, None, None] + b[:, None, None])
    return jnp.max(y, axis=-1)


def _head_body(feat_ref, wh_ref, bh_ref, out_ref):
    out_ref[...] = feat_ref[...] @ wh_ref[...] + bh_ref[...]


def kernel(point_coords, point_features, W1, g1, b1, W2, g2, b2, W3, g3, b3, W4, g4, b4, W5, g5, b5, Wf, gbn, bbn, Wh, bh):
    k = 20
    intensity = point_features[:, 3:4]
    x = jnp.concatenate([point_coords, intensity], axis=1).T
    x1 = _edge_conv(x, W1, g1, b1, k)
    x2 = _edge_conv(x1, W2, g2, b2, k)
    x3 = _edge_conv(x2, W3, g3, b3, k)
    x4 = _edge_conv(x3, W4, g4, b4, k)
    xc = jnp.concatenate([x1, x2, x3, x4], axis=0)
    x5 = _lrelu(jnp.einsum('oc,cn->on', W5, xc) * g5[:, None] + b5[:, None])
    feat = jnp.einsum('oc,cn->on', Wf, x5)
    feat = feat * gbn[:, None] + bbn[:, None]
    feat = feat.T
    logits = pl.pallas_call(
        _head_body,
        out_shape=jax.ShapeDtypeStruct((feat.shape[0], Wh.shape[1]), jnp.float32),
    )(feat, Wh, bh[None, :])
    return logits


# speedup vs baseline: 7.9994x; 7.9994x over previous
import functools

import jax
import jax.numpy as jnp
from jax import lax
from jax.experimental import pallas as pl
from jax.experimental.pallas import tpu as pltpu
from jax.experimental.pallas import tpu_sc as plsc

N = 5000
NP = 5120
K = 20
CP = 128


def _knn_body(xb_ref, xa_ref, idx_ref):
    xb = xb_ref[...]
    xa = xa_ref[...]
    inner = lax.dot_general(xb.astype(jnp.bfloat16), xa.astype(jnp.bfloat16),
                            (((1,), (1,)), ((), ())),
                            preferred_element_type=jnp.float32)
    xnb = jnp.sum(xb * xb, axis=1)
    xna = jnp.sum(xa * xa, axis=1)
    pd = (xnb[:, None] + xna[None, :]) - 2.0 * inner
    s = -pd
    col = lax.broadcasted_iota(jnp.int32, s.shape, 1)
    s = jnp.where(col < N, s, -jnp.inf)
    outs = []
    for _ in range(K):
        m = jnp.max(s, axis=1, keepdims=True)
        am = jnp.min(jnp.where(s == m, col, NP), axis=1, keepdims=True)
        outs.append(am)
        s = jnp.where(col == am, -jnp.inf, s)
    idx_ref[...] = jnp.concatenate(outs, axis=1)


def _topk(x_t, bn=256):
    return pl.pallas_call(
        _knn_body,
        grid=(NP // bn,),
        in_specs=[pl.BlockSpec((bn, CP), lambda i: (i, 0)),
                  pl.BlockSpec((NP, CP), lambda i: (0, 0))],
        out_specs=pl.BlockSpec((bn, K), lambda i: (i, 0)),
        out_shape=jax.ShapeDtypeStruct((NP, K), jnp.int32),
    )(x_t, x_t)


def _gather(x_t, idx_flat):
    d = x_t.shape[1]
    info = plsc.get_sparse_core_info()
    nc, ns = info.num_cores, info.num_subcores
    nw = nc * ns
    ppw = NP // nw
    p = 16
    steps = ppw // p
    mesh = plsc.VectorSubcoreMesh(core_axis_name="c", subcore_axis_name="s")

    @functools.partial(
        pl.kernel, mesh=mesh,
        out_type=jax.ShapeDtypeStruct((NP * K, d), jnp.float32),
        scratch_types=[
            pltpu.VMEM((p * K,), jnp.int32),
            pltpu.VMEM((p * K, d), jnp.float32),
            pltpu.SemaphoreType.DMA,
        ],
    )
    def body(x_hbm, idx_hbm, out_hbm, idx_v, rows_v, sem):
        wid = lax.axis_index("s") * nc + lax.axis_index("c")

        def step(si, carry):
            base = (wid * ppw + si * p) * K
            pltpu.sync_copy(idx_hbm.at[pl.ds(base, p * K)], idx_v)
            pltpu.async_copy(x_hbm.at[idx_v], rows_v, sem).wait()
            pltpu.sync_copy(rows_v, out_hbm.at[pl.ds(base, p * K)])
            return carry

        lax.fori_loop(0, steps, step, 0)

    return body(x_t, idx_flat)


def _conv_body(nb_ref, ctr_ref, wcat_ref, g_ref, b_ref, y_ref):
    nb = nb_ref[...]
    ctr = ctr_ref[...]
    bn = ctr.shape[0]
    nb3 = nb.reshape(bn, K, CP)
    diff = nb3 - ctr[:, None, :]
    ctr_rep = jnp.broadcast_to(ctr[:, None, :], (bn, K, CP))
    ef = jnp.concatenate([diff, ctr_rep], axis=2).reshape(bn * K, 2 * CP)
    y = jnp.dot(ef.astype(jnp.bfloat16), wcat_ref[...].astype(jnp.bfloat16),
                preferred_element_type=jnp.float32)
    y = y * g_ref[...] + b_ref[...]
    y = jnp.where(y > 0, y, 0.2 * y)
    y_ref[...] = jnp.max(y.reshape(bn, K, y.shape[-1]), axis=1)


def _conv(nb, x_t, wcat, gp, bp, bn=256):
    op = wcat.shape[1]
    return pl.pallas_call(
        _conv_body,
        grid=(NP // bn,),
        in_specs=[pl.BlockSpec((bn * K, CP), lambda i: (i, 0)),
                  pl.BlockSpec((bn, CP), lambda i: (i, 0)),
                  pl.BlockSpec((2 * CP, op), lambda i: (0, 0)),
                  pl.BlockSpec((1, op), lambda i: (0, 0)),
                  pl.BlockSpec((1, op), lambda i: (0, 0))],
        out_specs=pl.BlockSpec((bn, op), lambda i: (i, 0)),
        out_shape=jax.ShapeDtypeStruct((NP, op), jnp.float32),
    )(nb, x_t, wcat, gp, bp)


def _head_body(xc_ref, w5t_ref, g5_ref, b5_ref, wft_ref, gbn_ref, bbn_ref,
               wh_ref, bh_ref, out_ref):
    xc = xc_ref[...].astype(jnp.bfloat16)
    x5 = jnp.dot(xc, w5t_ref[...].astype(jnp.bfloat16),
                 preferred_element_type=jnp.float32)
    x5 = x5 * g5_ref[...] + b5_ref[...]
    x5 = jnp.where(x5 > 0, x5, 0.2 * x5)
    feat = jnp.dot(x5.astype(jnp.bfloat16), wft_ref[...].astype(jnp.bfloat16),
                   preferred_element_type=jnp.float32)
    feat = feat * gbn_ref[...] + bbn_ref[...]
    out_ref[...] = (jnp.dot(feat.astype(jnp.bfloat16),
                            wh_ref[...].astype(jnp.bfloat16),
                            preferred_element_type=jnp.float32) + bh_ref[...])


def _head(xc, W5, g5, b5, Wf, gbn, bbn, Wh, bh, bh_rows=1024):
    nh = Wh.shape[1]
    return pl.pallas_call(
        _head_body,
        grid=(NP // bh_rows,),
        in_specs=[
            pl.BlockSpec((bh_rows, 512), lambda i: (i, 0)),
            pl.BlockSpec((512, 1024), lambda i: (0, 0)),
            pl.BlockSpec((1, 1024), lambda i: (0, 0)),
            pl.BlockSpec((1, 1024), lambda i: (0, 0)),
            pl.BlockSpec((1024, 512), lambda i: (0, 0)),
            pl.BlockSpec((1, 512), lambda i: (0, 0)),
            pl.BlockSpec((1, 512), lambda i: (0, 0)),
            pl.BlockSpec((512, nh), lambda i: (0, 0)),
            pl.BlockSpec((1, nh), lambda i: (0, 0)),
        ],
        out_specs=pl.BlockSpec((bh_rows, nh), lambda i: (i, 0)),
        out_shape=jax.ShapeDtypeStruct((NP, nh), jnp.float32),
    )(xc, W5.T, g5[None, :], b5[None, :], Wf.T, gbn[None, :], bbn[None, :],
      Wh, bh[None, :])


def _edge_layer(x_t, c, W, g, b):
    idx = _topk(x_t)
    nb = _gather(x_t, idx.reshape(-1))
    o = W.shape[0]
    op = ((o + CP - 1) // CP) * CP
    wcat = jnp.zeros((2 * CP, op), jnp.float32)
    wcat = wcat.at[:c, :o].set(W[:, :c].T)
    wcat = wcat.at[CP:CP + c, :o].set(W[:, c:].T)
    gp = jnp.zeros((1, op), jnp.float32).at[:, :o].set(g[None, :])
    bp = jnp.zeros((1, op), jnp.float32).at[:, :o].set(b[None, :])
    return _conv(nb, x_t, wcat, gp, bp)


def kernel(point_coords, point_features, W1, g1, b1, W2, g2, b2, W3, g3, b3,
           W4, g4, b4, W5, g5, b5, Wf, gbn, bbn, Wh, bh):
    intensity = point_features[:, 3:4]
    x0 = jnp.concatenate([point_coords, intensity], axis=1)
    x_t = jnp.zeros((NP, CP), jnp.float32).at[:N, :4].set(x0)
    x1 = _edge_layer(x_t, 4, W1, g1, b1)
    x2 = _edge_layer(x1, 64, W2, g2, b2)
    x3 = _edge_layer(x2, 64, W3, g3, b3)
    x4 = _edge_layer(x3, 128, W4, g4, b4)
    xc = jnp.concatenate([x1[:, :64], x2[:, :64], x3, x4], axis=1)
    logits = _head(xc, W5, g5, b5, Wf, gbn, bbn, Wh, bh)
    return logits[:N]


# speedup vs baseline: 8.0080x; 1.0011x over previous
import functools

import jax
import jax.numpy as jnp
from jax import lax
from jax.experimental import pallas as pl
from jax.experimental.pallas import tpu as pltpu
from jax.experimental.pallas import tpu_sc as plsc

N = 5000
NP = 5120
K = 20
CP = 128


def _knn_body(xb_ref, xa_ref, idx_ref):
    xb = xb_ref[...]
    xa = xa_ref[...]
    inner = lax.dot_general(xb.astype(jnp.bfloat16), xa.astype(jnp.bfloat16),
                            (((1,), (1,)), ((), ())),
                            preferred_element_type=jnp.float32)
    xnb = jnp.sum(xb * xb, axis=1)
    xna = jnp.sum(xa * xa, axis=1)
    s = 2.0 * inner - (xnb[:, None] + xna[None, :])
    col = lax.broadcasted_iota(jnp.int32, s.shape, 1)
    s = jnp.where(col < N, s, -jnp.inf)
    outs = []
    for it in range(K):
        m = jnp.max(s, axis=1, keepdims=True)
        am = jnp.min(jnp.where(s == m, col, NP), axis=1, keepdims=True)
        outs.append(am)
        if it + 1 < K:
            s = jnp.where(col == am, -jnp.inf, s)
    idx_ref[...] = jnp.concatenate(outs, axis=1)


def _topk(x_t, bn=256):
    return pl.pallas_call(
        _knn_body,
        grid=(NP // bn,),
        in_specs=[pl.BlockSpec((bn, CP), lambda i: (i, 0)),
                  pl.BlockSpec((NP, CP), lambda i: (0, 0))],
        out_specs=pl.BlockSpec((bn, K), lambda i: (i, 0)),
        out_shape=jax.ShapeDtypeStruct((NP, K), jnp.int32),
    )(x_t, x_t)


def _gather(x_t, idx_flat):
    d = x_t.shape[1]
    info = plsc.get_sparse_core_info()
    nc, ns = info.num_cores, info.num_subcores
    nw = nc * ns
    ppw = NP // nw
    p = 16
    steps = ppw // p
    mesh = plsc.VectorSubcoreMesh(core_axis_name="c", subcore_axis_name="s")

    @functools.partial(
        pl.kernel, mesh=mesh,
        out_type=jax.ShapeDtypeStruct((NP * K, d), jnp.float32),
        scratch_types=[
            pltpu.VMEM((p * K,), jnp.int32),
            pltpu.VMEM((p * K, d), jnp.float32),
            pltpu.SemaphoreType.DMA,
        ],
    )
    def body(x_hbm, idx_hbm, out_hbm, idx_v, rows_v, sem):
        wid = lax.axis_index("s") * nc + lax.axis_index("c")

        def step(si, carry):
            base = (wid * ppw + si * p) * K
            pltpu.sync_copy(idx_hbm.at[pl.ds(base, p * K)], idx_v)
            pltpu.async_copy(x_hbm.at[idx_v], rows_v, sem).wait()
            pltpu.sync_copy(rows_v, out_hbm.at[pl.ds(base, p * K)])
            return carry

        lax.fori_loop(0, steps, step, 0)

    return body(x_t, idx_flat)


def _conv_body(nb_ref, ctr_ref, wcat_ref, g_ref, b_ref, y_ref):
    nb = nb_ref[...]
    ctr = ctr_ref[...]
    bn = ctr.shape[0]
    nb3 = nb.reshape(bn, K, CP)
    diff = nb3 - ctr[:, None, :]
    ctr_rep = jnp.broadcast_to(ctr[:, None, :], (bn, K, CP))
    ef = jnp.concatenate([diff, ctr_rep], axis=2).reshape(bn * K, 2 * CP)
    y = jnp.dot(ef.astype(jnp.bfloat16), wcat_ref[...].astype(jnp.bfloat16),
                preferred_element_type=jnp.float32)
    y = y * g_ref[...] + b_ref[...]
    y = jnp.where(y > 0, y, 0.2 * y)
    y_ref[...] = jnp.max(y.reshape(bn, K, y.shape[-1]), axis=1)


def _conv(nb, x_t, wcat, gp, bp, bn=256):
    op = wcat.shape[1]
    return pl.pallas_call(
        _conv_body,
        grid=(NP // bn,),
        in_specs=[pl.BlockSpec((bn * K, CP), lambda i: (i, 0)),
                  pl.BlockSpec((bn, CP), lambda i: (i, 0)),
                  pl.BlockSpec((2 * CP, op), lambda i: (0, 0)),
                  pl.BlockSpec((1, op), lambda i: (0, 0)),
                  pl.BlockSpec((1, op), lambda i: (0, 0))],
        out_specs=pl.BlockSpec((bn, op), lambda i: (i, 0)),
        out_shape=jax.ShapeDtypeStruct((NP, op), jnp.float32),
    )(nb, x_t, wcat, gp, bp)


def _head_body(xc_ref, w5t_ref, g5_ref, b5_ref, wft_ref, gbn_ref, bbn_ref,
               wh_ref, bh_ref, out_ref):
    xc = xc_ref[...].astype(jnp.bfloat16)
    x5 = jnp.dot(xc, w5t_ref[...].astype(jnp.bfloat16),
                 preferred_element_type=jnp.float32)
    x5 = x5 * g5_ref[...] + b5_ref[...]
    x5 = jnp.where(x5 > 0, x5, 0.2 * x5)
    feat = jnp.dot(x5.astype(jnp.bfloat16), wft_ref[...].astype(jnp.bfloat16),
                   preferred_element_type=jnp.float32)
    feat = feat * gbn_ref[...] + bbn_ref[...]
    out_ref[...] = (jnp.dot(feat.astype(jnp.bfloat16),
                            wh_ref[...].astype(jnp.bfloat16),
                            preferred_element_type=jnp.float32) + bh_ref[...])


def _head(xc, W5, g5, b5, Wf, gbn, bbn, Wh, bh, bh_rows=1024):
    nh = Wh.shape[1]
    return pl.pallas_call(
        _head_body,
        grid=(NP // bh_rows,),
        in_specs=[
            pl.BlockSpec((bh_rows, 512), lambda i: (i, 0)),
            pl.BlockSpec((512, 1024), lambda i: (0, 0)),
            pl.BlockSpec((1, 1024), lambda i: (0, 0)),
            pl.BlockSpec((1, 1024), lambda i: (0, 0)),
            pl.BlockSpec((1024, 512), lambda i: (0, 0)),
            pl.BlockSpec((1, 512), lambda i: (0, 0)),
            pl.BlockSpec((1, 512), lambda i: (0, 0)),
            pl.BlockSpec((512, nh), lambda i: (0, 0)),
            pl.BlockSpec((1, nh), lambda i: (0, 0)),
        ],
        out_specs=pl.BlockSpec((bh_rows, nh), lambda i: (i, 0)),
        out_shape=jax.ShapeDtypeStruct((NP, nh), jnp.float32),
    )(xc, W5.T, g5[None, :], b5[None, :], Wf.T, gbn[None, :], bbn[None, :],
      Wh, bh[None, :])


def _edge_layer(x_t, c, W, g, b):
    idx = _topk(x_t)
    nb = _gather(x_t, idx.reshape(-1))
    o = W.shape[0]
    op = ((o + CP - 1) // CP) * CP
    wcat = jnp.zeros((2 * CP, op), jnp.float32)
    wcat = wcat.at[:c, :o].set(W[:, :c].T)
    wcat = wcat.at[CP:CP + c, :o].set(W[:, c:].T)
    gp = jnp.zeros((1, op), jnp.float32).at[:, :o].set(g[None, :])
    bp = jnp.zeros((1, op), jnp.float32).at[:, :o].set(b[None, :])
    return _conv(nb, x_t, wcat, gp, bp)


def kernel(point_coords, point_features, W1, g1, b1, W2, g2, b2, W3, g3, b3,
           W4, g4, b4, W5, g5, b5, Wf, gbn, bbn, Wh, bh):
    intensity = point_features[:, 3:4]
    x0 = jnp.concatenate([point_coords, intensity], axis=1)
    x_t = jnp.zeros((NP, CP), jnp.float32).at[:N, :4].set(x0)
    x1 = _edge_layer(x_t, 4, W1, g1, b1)
    x2 = _edge_layer(x1, 64, W2, g2, b2)
    x3 = _edge_layer(x2, 64, W3, g3, b3)
    x4 = _edge_layer(x3, 128, W4, g4, b4)
    xc = jnp.concatenate([x1[:, :64], x2[:, :64], x3, x4], axis=1)
    logits = _head(xc, W5, g5, b5, Wf, gbn, bbn, Wh, bh)
    return logits[:N]


# speedup vs baseline: 8.1171x; 1.0136x over previous
import functools

import jax
import jax.numpy as jnp
from jax import lax
from jax.experimental import pallas as pl
from jax.experimental.pallas import tpu as pltpu
from jax.experimental.pallas import tpu_sc as plsc

N = 5000
NP = 5120
NH = NP // 2
K = 20
CP = 128


def _knn_body(xb_ref, xa_ref, idx_ref):
    xb = xb_ref[...]
    xa = xa_ref[...]
    inner = lax.dot_general(xb.astype(jnp.bfloat16), xa.astype(jnp.bfloat16),
                            (((1,), (1,)), ((), ())),
                            preferred_element_type=jnp.float32)
    xnb = jnp.sum(xb * xb, axis=1)
    xna = jnp.sum(xa * xa, axis=1)
    s = 2.0 * inner - (xnb[:, None] + xna[None, :])
    col = lax.broadcasted_iota(jnp.int32, s.shape, 1)
    s = jnp.where(col < N, s, -jnp.inf)
    outs = []
    for it in range(K):
        m = jnp.max(s, axis=1, keepdims=True)
        am = jnp.min(jnp.where(s == m, col, NP), axis=1, keepdims=True)
        outs.append(am)
        if it + 1 < K:
            s = jnp.where(col == am, -jnp.inf, s)
    idx_ref[...] = jnp.concatenate(outs, axis=1)


def _topk(x_t, row0, nrows, bn=256):
    off = row0 // bn
    return pl.pallas_call(
        _knn_body,
        grid=(nrows // bn,),
        in_specs=[pl.BlockSpec((bn, CP), lambda i: (i + off, 0)),
                  pl.BlockSpec((NP, CP), lambda i: (0, 0))],
        out_specs=pl.BlockSpec((bn, K), lambda i: (i, 0)),
        out_shape=jax.ShapeDtypeStruct((nrows, K), jnp.int32),
    )(x_t, x_t)


def _gather(x_t, idx_flat, nrows):
    d = x_t.shape[1]
    info = plsc.get_sparse_core_info()
    nc, ns = info.num_cores, info.num_subcores
    nw = nc * ns
    ppw = nrows // nw
    p = 16
    steps = ppw // p
    mesh = plsc.VectorSubcoreMesh(core_axis_name="c", subcore_axis_name="s")

    @functools.partial(
        pl.kernel, mesh=mesh,
        out_type=jax.ShapeDtypeStruct((nrows * K, d), jnp.float32),
        scratch_types=[
            pltpu.VMEM((p * K,), jnp.int32),
            pltpu.VMEM((p * K, d), jnp.float32),
            pltpu.SemaphoreType.DMA,
        ],
    )
    def body(x_hbm, idx_hbm, out_hbm, idx_v, rows_v, sem):
        wid = lax.axis_index("s") * nc + lax.axis_index("c")

        def step(si, carry):
            base = (wid * ppw + si * p) * K
            pltpu.sync_copy(idx_hbm.at[pl.ds(base, p * K)], idx_v)
            pltpu.async_copy(x_hbm.at[idx_v], rows_v, sem).wait()
            pltpu.sync_copy(rows_v, out_hbm.at[pl.ds(base, p * K)])
            return carry

        lax.fori_loop(0, steps, step, 0)

    return body(x_t, idx_flat)


def _conv_body(nb_ref, ctr_ref, wcat_ref, g_ref, b_ref, y_ref):
    nb = nb_ref[...]
    ctr = ctr_ref[...]
    bn = ctr.shape[0]
    nb3 = nb.reshape(bn, K, CP)
    diff = nb3 - ctr[:, None, :]
    ctr_rep = jnp.broadcast_to(ctr[:, None, :], (bn, K, CP))
    ef = jnp.concatenate([diff, ctr_rep], axis=2).reshape(bn * K, 2 * CP)
    y = jnp.dot(ef.astype(jnp.bfloat16), wcat_ref[...].astype(jnp.bfloat16),
                preferred_element_type=jnp.float32)
    y = y * g_ref[...] + b_ref[...]
    y = jnp.where(y > 0, y, 0.2 * y)
    y_ref[...] = jnp.max(y.reshape(bn, K, y.shape[-1]), axis=1)


def _conv(nb, x_t, wcat, gp, bp, row0, nrows, bn=256):
    op = wcat.shape[1]
    off = row0 // bn
    return pl.pallas_call(
        _conv_body,
        grid=(nrows // bn,),
        in_specs=[pl.BlockSpec((bn * K, CP), lambda i: (i, 0)),
                  pl.BlockSpec((bn, CP), lambda i: (i + off, 0)),
                  pl.BlockSpec((2 * CP, op), lambda i: (0, 0)),
                  pl.BlockSpec((1, op), lambda i: (0, 0)),
                  pl.BlockSpec((1, op), lambda i: (0, 0))],
        out_specs=pl.BlockSpec((bn, op), lambda i: (i, 0)),
        out_shape=jax.ShapeDtypeStruct((nrows, op), jnp.float32),
    )(nb, x_t, wcat, gp, bp)


def _head_body(xc_ref, w5t_ref, g5_ref, b5_ref, wft_ref, gbn_ref, bbn_ref,
               wh_ref, bh_ref, out_ref):
    xc = xc_ref[...].astype(jnp.bfloat16)
    x5 = jnp.dot(xc, w5t_ref[...].astype(jnp.bfloat16),
                 preferred_element_type=jnp.float32)
    x5 = x5 * g5_ref[...] + b5_ref[...]
    x5 = jnp.where(x5 > 0, x5, 0.2 * x5)
    feat = jnp.dot(x5.astype(jnp.bfloat16), wft_ref[...].astype(jnp.bfloat16),
                   preferred_element_type=jnp.float32)
    feat = feat * gbn_ref[...] + bbn_ref[...]
    out_ref[...] = (jnp.dot(feat.astype(jnp.bfloat16),
                            wh_ref[...].astype(jnp.bfloat16),
                            preferred_element_type=jnp.float32) + bh_ref[...])


def _head(xc, W5, g5, b5, Wf, gbn, bbn, Wh, bh, bh_rows=1024):
    nh = Wh.shape[1]
    return pl.pallas_call(
        _head_body,
        grid=(NP // bh_rows,),
        in_specs=[
            pl.BlockSpec((bh_rows, 512), lambda i: (i, 0)),
            pl.BlockSpec((512, 1024), lambda i: (0, 0)),
            pl.BlockSpec((1, 1024), lambda i: (0, 0)),
            pl.BlockSpec((1, 1024), lambda i: (0, 0)),
            pl.BlockSpec((1024, 512), lambda i: (0, 0)),
            pl.BlockSpec((1, 512), lambda i: (0, 0)),
            pl.BlockSpec((1, 512), lambda i: (0, 0)),
            pl.BlockSpec((512, nh), lambda i: (0, 0)),
            pl.BlockSpec((1, nh), lambda i: (0, 0)),
        ],
        out_specs=pl.BlockSpec((bh_rows, nh), lambda i: (i, 0)),
        out_shape=jax.ShapeDtypeStruct((NP, nh), jnp.float32),
    )(xc, W5.T, g5[None, :], b5[None, :], Wf.T, gbn[None, :], bbn[None, :],
      Wh, bh[None, :])


def _edge_layer(x_t, c, W, g, b):
    o = W.shape[0]
    op = ((o + CP - 1) // CP) * CP
    wcat = jnp.zeros((2 * CP, op), jnp.float32)
    wcat = wcat.at[:c, :o].set(W[:, :c].T)
    wcat = wcat.at[CP:CP + c, :o].set(W[:, c:].T)
    gp = jnp.zeros((1, op), jnp.float32).at[:, :o].set(g[None, :])
    bp = jnp.zeros((1, op), jnp.float32).at[:, :o].set(b[None, :])
    ia = _topk(x_t, 0, NH)
    nba = _gather(x_t, ia.reshape(-1), NH)
    ib = _topk(x_t, NH, NH)
    nbb = _gather(x_t, ib.reshape(-1), NH)
    ya = _conv(nba, x_t, wcat, gp, bp, 0, NH)
    yb = _conv(nbb, x_t, wcat, gp, bp, NH, NH)
    return jnp.concatenate([ya, yb], axis=0)


def kernel(point_coords, point_features, W1, g1, b1, W2, g2, b2, W3, g3, b3,
           W4, g4, b4, W5, g5, b5, Wf, gbn, bbn, Wh, bh):
    intensity = point_features[:, 3:4]
    x0 = jnp.concatenate([point_coords, intensity], axis=1)
    x_t = jnp.zeros((NP, CP), jnp.float32).at[:N, :4].set(x0)
    x1 = _edge_layer(x_t, 4, W1, g1, b1)
    x2 = _edge_layer(x1, 64, W2, g2, b2)
    x3 = _edge_layer(x2, 64, W3, g3, b3)
    x4 = _edge_layer(x3, 128, W4, g4, b4)
    xc = jnp.concatenate([x1[:, :64], x2[:, :64], x3, x4], axis=1)
    logits = _head(xc, W5, g5, b5, Wf, gbn, bbn, Wh, bh)
    return logits[:N]


# speedup vs baseline: 9.7988x; 1.2072x over previous
import functools

import jax
import jax.numpy as jnp
from jax import lax
from jax.experimental import pallas as pl
from jax.experimental.pallas import tpu as pltpu
from jax.experimental.pallas import tpu_sc as plsc

N = 5000
NP = 5120
NH = NP // 2
K = 20
CP = 128


def _knn_body(xb_ref, xa_ref, idx_ref):
    xb = xb_ref[...]
    xa = xa_ref[...]
    inner = lax.dot_general(xb.astype(jnp.bfloat16), xa.astype(jnp.bfloat16),
                            (((1,), (1,)), ((), ())),
                            preferred_element_type=jnp.float32)
    xnb = jnp.sum(xb * xb, axis=1)
    xna = jnp.sum(xa * xa, axis=1)
    s = 2.0 * inner - (xnb[:, None] + xna[None, :])
    colf = lax.broadcasted_iota(jnp.int32, s.shape, 1).astype(jnp.float32)
    s = jnp.where(colf < N, s, -jnp.inf)
    outs = []
    for it in range(K):
        m = jnp.max(s, axis=1, keepdims=True)
        amf = jnp.min(jnp.where(s == m, colf, jnp.float32(3e38)),
                      axis=1, keepdims=True)
        outs.append(amf)
        if it + 1 < K:
            s = jnp.where(colf == amf, -jnp.inf, s)
    idx_ref[...] = jnp.concatenate(outs, axis=1).astype(jnp.int32)


def _topk(x_t, row0, nrows, bn=256):
    off = row0 // bn
    return pl.pallas_call(
        _knn_body,
        grid=(nrows // bn,),
        in_specs=[pl.BlockSpec((bn, CP), lambda i: (i + off, 0)),
                  pl.BlockSpec((NP, CP), lambda i: (0, 0))],
        out_specs=pl.BlockSpec((bn, K), lambda i: (i, 0)),
        out_shape=jax.ShapeDtypeStruct((nrows, K), jnp.int32),
    )(x_t, x_t)


def _gather(x_t, idx_flat, nrows):
    d = x_t.shape[1]
    info = plsc.get_sparse_core_info()
    nc, ns = info.num_cores, info.num_subcores
    nw = nc * ns
    ppw = nrows // nw
    p = 16
    steps = ppw // p
    mesh = plsc.VectorSubcoreMesh(core_axis_name="c", subcore_axis_name="s")

    @functools.partial(
        pl.kernel, mesh=mesh,
        out_type=jax.ShapeDtypeStruct((nrows * K, d), jnp.float32),
        scratch_types=[
            pltpu.VMEM((p * K,), jnp.int32),
            pltpu.VMEM((p * K, d), jnp.float32),
            pltpu.SemaphoreType.DMA,
        ],
    )
    def body(x_hbm, idx_hbm, out_hbm, idx_v, rows_v, sem):
        wid = lax.axis_index("s") * nc + lax.axis_index("c")

        def step(si, carry):
            base = (wid * ppw + si * p) * K
            pltpu.sync_copy(idx_hbm.at[pl.ds(base, p * K)], idx_v)
            pltpu.async_copy(x_hbm.at[idx_v], rows_v, sem).wait()
            pltpu.sync_copy(rows_v, out_hbm.at[pl.ds(base, p * K)])
            return carry

        lax.fori_loop(0, steps, step, 0)

    return body(x_t, idx_flat)


def _conv_body(nb_ref, ctr_ref, wcat_ref, g_ref, b_ref, y_ref):
    nb = nb_ref[...]
    ctr = ctr_ref[...]
    bn = ctr.shape[0]
    nb3 = nb.reshape(bn, K, CP)
    diff = nb3 - ctr[:, None, :]
    ctr_rep = jnp.broadcast_to(ctr[:, None, :], (bn, K, CP))
    ef = jnp.concatenate([diff, ctr_rep], axis=2).reshape(bn * K, 2 * CP)
    y = jnp.dot(ef.astype(jnp.bfloat16), wcat_ref[...].astype(jnp.bfloat16),
                preferred_element_type=jnp.float32)
    y = y * g_ref[...] + b_ref[...]
    y = jnp.where(y > 0, y, 0.2 * y)
    y_ref[...] = jnp.max(y.reshape(bn, K, y.shape[-1]), axis=1)


def _conv(nb, x_t, wcat, gp, bp, row0, nrows, bn=256):
    op = wcat.shape[1]
    off = row0 // bn
    return pl.pallas_call(
        _conv_body,
        grid=(nrows // bn,),
        in_specs=[pl.BlockSpec((bn * K, CP), lambda i: (i, 0)),
                  pl.BlockSpec((bn, CP), lambda i: (i + off, 0)),
                  pl.BlockSpec((2 * CP, op), lambda i: (0, 0)),
                  pl.BlockSpec((1, op), lambda i: (0, 0)),
                  pl.BlockSpec((1, op), lambda i: (0, 0))],
        out_specs=pl.BlockSpec((bn, op), lambda i: (i, 0)),
        out_shape=jax.ShapeDtypeStruct((nrows, op), jnp.float32),
    )(nb, x_t, wcat, gp, bp)


def _head_body(xc_ref, w5t_ref, g5_ref, b5_ref, wft_ref, gbn_ref, bbn_ref,
               wh_ref, bh_ref, out_ref):
    xc = xc_ref[...].astype(jnp.bfloat16)
    x5 = jnp.dot(xc, w5t_ref[...].astype(jnp.bfloat16),
                 preferred_element_type=jnp.float32)
    x5 = x5 * g5_ref[...] + b5_ref[...]
    x5 = jnp.where(x5 > 0, x5, 0.2 * x5)
    feat = jnp.dot(x5.astype(jnp.bfloat16), wft_ref[...].astype(jnp.bfloat16),
                   preferred_element_type=jnp.float32)
    feat = feat * gbn_ref[...] + bbn_ref[...]
    out_ref[...] = (jnp.dot(feat.astype(jnp.bfloat16),
                            wh_ref[...].astype(jnp.bfloat16),
                            preferred_element_type=jnp.float32) + bh_ref[...])


def _head(xc, W5, g5, b5, Wf, gbn, bbn, Wh, bh, bh_rows=1024):
    nh = Wh.shape[1]
    return pl.pallas_call(
        _head_body,
        grid=(NP // bh_rows,),
        in_specs=[
            pl.BlockSpec((bh_rows, 512), lambda i: (i, 0)),
            pl.BlockSpec((512, 1024), lambda i: (0, 0)),
            pl.BlockSpec((1, 1024), lambda i: (0, 0)),
            pl.BlockSpec((1, 1024), lambda i: (0, 0)),
            pl.BlockSpec((1024, 512), lambda i: (0, 0)),
            pl.BlockSpec((1, 512), lambda i: (0, 0)),
            pl.BlockSpec((1, 512), lambda i: (0, 0)),
            pl.BlockSpec((512, nh), lambda i: (0, 0)),
            pl.BlockSpec((1, nh), lambda i: (0, 0)),
        ],
        out_specs=pl.BlockSpec((bh_rows, nh), lambda i: (i, 0)),
        out_shape=jax.ShapeDtypeStruct((NP, nh), jnp.float32),
    )(xc, W5.T, g5[None, :], b5[None, :], Wf.T, gbn[None, :], bbn[None, :],
      Wh, bh[None, :])


def _edge_layer(x_t, c, W, g, b):
    o = W.shape[0]
    op = ((o + CP - 1) // CP) * CP
    wcat = jnp.zeros((2 * CP, op), jnp.float32)
    wcat = wcat.at[:c, :o].set(W[:, :c].T)
    wcat = wcat.at[CP:CP + c, :o].set(W[:, c:].T)
    gp = jnp.zeros((1, op), jnp.float32).at[:, :o].set(g[None, :])
    bp = jnp.zeros((1, op), jnp.float32).at[:, :o].set(b[None, :])
    ia = _topk(x_t, 0, NH)
    nba = _gather(x_t, ia.reshape(-1), NH)
    ib = _topk(x_t, NH, NH)
    nbb = _gather(x_t, ib.reshape(-1), NH)
    ya = _conv(nba, x_t, wcat, gp, bp, 0, NH)
    yb = _conv(nbb, x_t, wcat, gp, bp, NH, NH)
    return jnp.concatenate([ya, yb], axis=0)


def kernel(point_coords, point_features, W1, g1, b1, W2, g2, b2, W3, g3, b3,
           W4, g4, b4, W5, g5, b5, Wf, gbn, bbn, Wh, bh):
    intensity = point_features[:, 3:4]
    x0 = jnp.concatenate([point_coords, intensity], axis=1)
    x_t = jnp.zeros((NP, CP), jnp.float32).at[:N, :4].set(x0)
    x1 = _edge_layer(x_t, 4, W1, g1, b1)
    x2 = _edge_layer(x1, 64, W2, g2, b2)
    x3 = _edge_layer(x2, 64, W3, g3, b3)
    x4 = _edge_layer(x3, 128, W4, g4, b4)
    xc = jnp.concatenate([x1[:, :64], x2[:, :64], x3, x4], axis=1)
    logits = _head(xc, W5, g5, b5, Wf, gbn, bbn, Wh, bh)
    return logits[:N]
